# Initial kernel scaffold; baseline (speedup 1.0000x reference)
import jax
import jax.numpy as jnp
from jax.experimental import pallas as pl


def kernel(x, i, W1, b1, W2, b2, W3, b3, W4, b4):
    raise NotImplementedError("write your pallas kernel here")


# speedup vs baseline: 4.8672x; 4.8672x over previous
import functools

import jax
import jax.numpy as jnp
from jax.experimental import pallas as pl

NUM_SEG = 128
STAT_W = 136


def _pick_block(n):
    for b in (2560, 1280, 640, 320, 160, 80, 40, 16, 8):
        if n % b == 0:
            return b
    return n


def _lrelu(a):
    return jnp.where(a >= 0, a, 0.01 * a)


def _onehot_t(i_ref, blk):
    ids = i_ref[...]
    return (jax.lax.broadcasted_iota(jnp.int32, (NUM_SEG, blk), 0) == ids
            ).astype(jnp.bfloat16)


def _seg_stats(mt, h, blk):
    hh = jnp.concatenate(
        [h.astype(jnp.bfloat16),
         (h * h).astype(jnp.bfloat16),
         jnp.ones((blk, 8), jnp.bfloat16)], axis=1)
    return jax.lax.dot_general(mt, hh, (((1,), (0,)), ((), ())),
                               preferred_element_type=jnp.float32)


def _finalize(s_ref):
    sums = s_ref[:, 0:64]
    sqs = s_ref[:, 64:128]
    cnt = s_ref[:, 128:129]
    mean = jnp.where(cnt > 0, sums / jnp.maximum(cnt, 1.0), 0.0)
    var = jnp.where(cnt > 1,
                    (sqs - cnt * mean * mean) / jnp.maximum(cnt - 1.0, 1.0),
                    0.0)
    std = jnp.sqrt(jnp.maximum(var, 0.0))
    rstd = 1.0 / (std + 1e-8)
    return mean, rstd


def _gather_rows(mt, table):
    hi = table.astype(jnp.bfloat16)
    lo = (table - hi.astype(jnp.float32)).astype(jnp.bfloat16)
    dims = (((0,), (0,)), ((), ()))
    return (jax.lax.dot_general(mt, hi, dims, preferred_element_type=jnp.float32)
            + jax.lax.dot_general(mt, lo, dims, preferred_element_type=jnp.float32))


def _normalize(h, mt, s_ref):
    mean, rstd = _finalize(s_ref)
    table = jnp.concatenate([mean * rstd, rstd], axis=1)
    r = _gather_rows(mt, table)
    return h * r[:, 64:128] - r[:, 0:64]


def _first_kernel(x_ref, i_ref, w_ref, b_ref, h_ref, s_ref, *, blk):
    g = pl.program_id(0)
    a = jax.lax.dot_general(x_ref[...], w_ref[...], (((1,), (1,)), ((), ())),
                            preferred_element_type=jnp.float32,
                            precision=jax.lax.Precision.HIGHEST)
    h = _lrelu(a + b_ref[...])
    h_ref[...] = h
    st = _seg_stats(_onehot_t(i_ref, blk), h, blk)

    @pl.when(g == 0)
    def _():
        s_ref[...] = st

    @pl.when(g > 0)
    def _():
        s_ref[...] += st


def _mid_kernel(h_ref, i_ref, s_ref, w_ref, b_ref, ho_ref, so_ref, *, blk):
    g = pl.program_id(0)
    mt = _onehot_t(i_ref, blk)
    z = _normalize(h_ref[...], mt, s_ref)
    a = jax.lax.dot_general(z, w_ref[...], (((1,), (1,)), ((), ())),
                            preferred_element_type=jnp.float32,
                            precision=jax.lax.Precision.HIGHEST)
    h = _lrelu(a + b_ref[...])
    ho_ref[...] = h
    st = _seg_stats(mt, h, blk)

    @pl.when(g == 0)
    def _():
        so_ref[...] = st

    @pl.when(g > 0)
    def _():
        so_ref[...] += st


def _last_kernel(h_ref, i_ref, s_ref, w_ref, b_ref, o_ref, *, blk):
    mt = _onehot_t(i_ref, blk)
    z = _normalize(h_ref[...], mt, s_ref)
    o_ref[...] = jax.lax.dot_general(z, w_ref[...], (((1,), (1,)), ((), ())),
                                     preferred_element_type=jnp.float32,
                                     precision=jax.lax.Precision.HIGHEST
                                     ) + b_ref[...]


def kernel(x, i, W1, b1, W2, b2, W3, b3, W4, b4):
    n, d = x.shape
    blk = _pick_block(n)
    nb = n // blk
    grid = (nb,)
    i_row = i.reshape(1, n)

    row_spec = lambda w: pl.BlockSpec((blk, w), lambda g: (g, 0))
    i_spec = pl.BlockSpec((1, blk), lambda g: (0, g))
    full = lambda a, b: pl.BlockSpec((a, b), lambda g: (0, 0))
    stat_shape = jax.ShapeDtypeStruct((NUM_SEG, STAT_W), jnp.float32)
    h_shape = jax.ShapeDtypeStruct((n, 64), jnp.float32)

    h1, s1 = pl.pallas_call(
        functools.partial(_first_kernel, blk=blk),
        grid=grid,
        in_specs=[row_spec(d), i_spec, full(64, d), full(1, 64)],
        out_specs=[row_spec(64), full(NUM_SEG, STAT_W)],
        out_shape=[h_shape, stat_shape],
    )(x, i_row, W1, b1.reshape(1, 64))

    mid = pl.pallas_call(
        functools.partial(_mid_kernel, blk=blk),
        grid=grid,
        in_specs=[row_spec(64), i_spec, full(NUM_SEG, STAT_W),
                  full(64, 64), full(1, 64)],
        out_specs=[row_spec(64), full(NUM_SEG, STAT_W)],
        out_shape=[h_shape, stat_shape],
    )
    h2, s2 = mid(h1, i_row, s1, W2, b2.reshape(1, 64))
    h3, s3 = mid(h2, i_row, s2, W3, b3.reshape(1, 64))

    out = pl.pallas_call(
        functools.partial(_last_kernel, blk=blk),
        grid=grid,
        in_specs=[row_spec(64), i_spec, full(NUM_SEG, STAT_W),
                  full(3, 64), full(1, 3)],
        out_specs=row_spec(3),
        out_shape=jax.ShapeDtypeStruct((n, 3), jnp.float32),
    )(h3, i_row, s3, W4, b4.reshape(1, 3))
    return out


# speedup vs baseline: 7.0007x; 1.4383x over previous
import functools

import jax
import jax.numpy as jnp
from jax.experimental import pallas as pl

NUM_SEG = 128
STAT_W = 136


def _pick_block(n):
    for b in (2560, 1280, 640, 320, 160, 80, 40, 16, 8):
        if n % b == 0:
            return b
    return n


def _lrelu(a):
    return jnp.where(a >= 0, a, 0.01 * a)


def _onehot_t(i_ref, blk):
    ids = i_ref[...]
    return (jax.lax.broadcasted_iota(jnp.int32, (NUM_SEG, blk), 0) == ids
            ).astype(jnp.bfloat16)


def _seg_stats(mt, h, blk):
    hh = jnp.concatenate(
        [h.astype(jnp.bfloat16),
         (h * h).astype(jnp.bfloat16),
         jnp.ones((blk, 8), jnp.bfloat16)], axis=1)
    return jax.lax.dot_general(mt, hh, (((1,), (0,)), ((), ())),
                               preferred_element_type=jnp.float32)


def _finalize(s_ref):
    sums = s_ref[:, 0:64]
    sqs = s_ref[:, 64:128]
    cnt = s_ref[:, 128:129]
    mean = jnp.where(cnt > 0, sums / jnp.maximum(cnt, 1.0), 0.0)
    var = jnp.where(cnt > 1,
                    (sqs - cnt * mean * mean) / jnp.maximum(cnt - 1.0, 1.0),
                    0.0)
    std = jnp.sqrt(jnp.maximum(var, 0.0))
    rstd = 1.0 / (std + 1e-8)
    return mean, rstd


def _gather_rows(mt, table):
    hi = table.astype(jnp.bfloat16)
    lo = (table - hi.astype(jnp.float32)).astype(jnp.bfloat16)
    dims = (((0,), (0,)), ((), ()))
    return (jax.lax.dot_general(mt, hi, dims, preferred_element_type=jnp.float32)
            + jax.lax.dot_general(mt, lo, dims, preferred_element_type=jnp.float32))


def _normalize(h, mt, s_ref):
    mean, rstd = _finalize(s_ref)
    table = jnp.concatenate([mean * rstd, rstd], axis=1)
    r = _gather_rows(mt, table)
    return h * r[:, 64:128] - r[:, 0:64]


def _first_kernel(x_ref, i_ref, w_ref, b_ref, h_ref, s_ref, *, blk):
    g = pl.program_id(0)
    a = jax.lax.dot_general(x_ref[...], w_ref[...], (((1,), (1,)), ((), ())),
                            preferred_element_type=jnp.float32)
    h = _lrelu(a + b_ref[...])
    h_ref[...] = h
    st = _seg_stats(_onehot_t(i_ref, blk), h, blk)

    @pl.when(g == 0)
    def _():
        s_ref[...] = st

    @pl.when(g > 0)
    def _():
        s_ref[...] += st


def _mid_kernel(h_ref, i_ref, s_ref, w_ref, b_ref, ho_ref, so_ref, *, blk):
    g = pl.program_id(0)
    mt = _onehot_t(i_ref, blk)
    z = _normalize(h_ref[...], mt, s_ref)
    a = jax.lax.dot_general(z, w_ref[...], (((1,), (1,)), ((), ())),
                            preferred_element_type=jnp.float32)
    h = _lrelu(a + b_ref[...])
    ho_ref[...] = h
    st = _seg_stats(mt, h, blk)

    @pl.when(g == 0)
    def _():
        so_ref[...] = st

    @pl.when(g > 0)
    def _():
        so_ref[...] += st


def _last_kernel(h_ref, i_ref, s_ref, w_ref, b_ref, o_ref, *, blk):
    mt = _onehot_t(i_ref, blk)
    z = _normalize(h_ref[...], mt, s_ref)
    o_ref[...] = jax.lax.dot_general(z, w_ref[...], (((1,), (1,)), ((), ())),
                                     preferred_element_type=jnp.float32
                                     ) + b_ref[...]


def kernel(x, i, W1, b1, W2, b2, W3, b3, W4, b4):
    n, d = x.shape
    blk = _pick_block(n)
    nb = n // blk
    grid = (nb,)
    i_row = i.reshape(1, n)

    row_spec = lambda w: pl.BlockSpec((blk, w), lambda g: (g, 0))
    i_spec = pl.BlockSpec((1, blk), lambda g: (0, g))
    full = lambda a, b: pl.BlockSpec((a, b), lambda g: (0, 0))
    stat_shape = jax.ShapeDtypeStruct((NUM_SEG, STAT_W), jnp.float32)
    h_shape = jax.ShapeDtypeStruct((n, 64), jnp.float32)

    h1, s1 = pl.pallas_call(
        functools.partial(_first_kernel, blk=blk),
        grid=grid,
        in_specs=[row_spec(d), i_spec, full(64, d), full(1, 64)],
        out_specs=[row_spec(64), full(NUM_SEG, STAT_W)],
        out_shape=[h_shape, stat_shape],
    )(x, i_row, W1, b1.reshape(1, 64))

    mid = pl.pallas_call(
        functools.partial(_mid_kernel, blk=blk),
        grid=grid,
        in_specs=[row_spec(64), i_spec, full(NUM_SEG, STAT_W),
                  full(64, 64), full(1, 64)],
        out_specs=[row_spec(64), full(NUM_SEG, STAT_W)],
        out_shape=[h_shape, stat_shape],
    )
    h2, s2 = mid(h1, i_row, s1, W2, b2.reshape(1, 64))
    h3, s3 = mid(h2, i_row, s2, W3, b3.reshape(1, 64))

    out = pl.pallas_call(
        functools.partial(_last_kernel, blk=blk),
        grid=grid,
        in_specs=[row_spec(64), i_spec, full(NUM_SEG, STAT_W),
                  full(3, 64), full(1, 3)],
        out_specs=row_spec(3),
        out_shape=jax.ShapeDtypeStruct((n, 3), jnp.float32),
    )(h3, i_row, s3, W4, b4.reshape(1, 3))
    return out
